# baseline (speedup 1.0000x reference)
import jax
import jax.numpy as jnp
from jax import lax
from jax.experimental import pallas as pl
from jax.experimental.pallas import tpu as pltpu
from jax.experimental.pallas import tpu_sc as plsc

_NC = 2
_NS = 16
_L = 16
_NW = _NC * _NS
_B = 16384
_BPW = _B // _NW
_TABLE = 1001


def _gather_body(tab_hbm, idx_hbm, out_hbm, tab_v, idx_v, out_v, sem_t, sem_i):
    wid = lax.axis_index("s") * _NC + lax.axis_index("c")
    base = wid * _BPW
    ct = pltpu.async_copy(tab_hbm, tab_v, sem_t)
    ci = pltpu.async_copy(idx_hbm.at[pl.ds(base, _BPW)], idx_v, sem_i)
    ct.wait()
    ci.wait()
    half = _BPW // 2

    def step(lo):
        def body(i, carry):
            off = lo + i * _L
            idx = idx_v[pl.ds(off, _L)]
            out_v[pl.ds(off, _L)] = plsc.load_gather(tab_v, [idx])
            return carry
        return body

    lax.fori_loop(0, half // _L, step(0), 0, unroll=1)
    co = pltpu.async_copy(
        out_v.at[pl.ds(0, half)], out_hbm.at[pl.ds(base, half)], sem_i
    )
    lax.fori_loop(0, half // _L, step(half), 0, unroll=1)
    pltpu.sync_copy(
        out_v.at[pl.ds(half, half)], out_hbm.at[pl.ds(base + half, half)]
    )
    co.wait()


def kernel(t, alpha, alpha_bar):
    del alpha
    mesh = plsc.VectorSubcoreMesh(core_axis_name="c", subcore_axis_name="s")
    f = pl.kernel(
        _gather_body,
        out_type=jax.ShapeDtypeStruct((_B,), jnp.float32),
        mesh=mesh,
        scratch_types=[
            pltpu.VMEM((_TABLE,), jnp.float32),
            pltpu.VMEM((_BPW,), jnp.int32),
            pltpu.VMEM((_BPW,), jnp.float32),
            pltpu.SemaphoreType.DMA,
            pltpu.SemaphoreType.DMA,
        ],
        compiler_params=pltpu.CompilerParams(
            needs_layout_passes=False,
            disable_bounds_checks=True,
            disable_semaphore_checks=True,
            skip_device_barrier=True,
        ),
    )
    return f(alpha_bar, t)


# speedup vs baseline: 1.0718x; 1.0718x over previous
import jax
import jax.numpy as jnp
from jax import lax
from jax.experimental import pallas as pl
from jax.experimental.pallas import tpu as pltpu
from jax.experimental.pallas import tpu_sc as plsc

_NC = 1
_NS = 16
_L = 16
_NW = _NC * _NS
_B = 16384
_BPW = _B // _NW
_TABLE = 1001


def _gather_body(tab_hbm, idx_hbm, out_hbm, tab_v, idx_v, out_v, sem_t, sem_i):
    wid = lax.axis_index("s") * _NC + lax.axis_index("c")
    base = wid * _BPW
    ct = pltpu.async_copy(tab_hbm, tab_v, sem_t)
    ci = pltpu.async_copy(idx_hbm.at[pl.ds(base, _BPW)], idx_v, sem_i)
    ct.wait()
    ci.wait()
    half = _BPW // 2

    def step(lo):
        def body(i, carry):
            off = lo + i * _L
            idx = idx_v[pl.ds(off, _L)]
            out_v[pl.ds(off, _L)] = plsc.load_gather(tab_v, [idx])
            return carry
        return body

    lax.fori_loop(0, half // _L, step(0), 0, unroll=4)
    co = pltpu.async_copy(
        out_v.at[pl.ds(0, half)], out_hbm.at[pl.ds(base, half)], sem_i
    )
    lax.fori_loop(0, half // _L, step(half), 0, unroll=4)
    pltpu.sync_copy(
        out_v.at[pl.ds(half, half)], out_hbm.at[pl.ds(base + half, half)]
    )
    co.wait()


def kernel(t, alpha, alpha_bar):
    del alpha
    mesh = plsc.VectorSubcoreMesh(core_axis_name="c", subcore_axis_name="s", num_cores=1)
    f = pl.kernel(
        _gather_body,
        out_type=jax.ShapeDtypeStruct((_B,), jnp.float32),
        mesh=mesh,
        scratch_types=[
            pltpu.VMEM((_TABLE,), jnp.float32),
            pltpu.VMEM((_BPW,), jnp.int32),
            pltpu.VMEM((_BPW,), jnp.float32),
            pltpu.SemaphoreType.DMA,
            pltpu.SemaphoreType.DMA,
        ],
        compiler_params=pltpu.CompilerParams(
            needs_layout_passes=False,
            disable_bounds_checks=True,
            disable_semaphore_checks=True,
            skip_device_barrier=True,
        ),
    )
    return f(alpha_bar, t)


# speedup vs baseline: 1.0805x; 1.0081x over previous
import jax
import jax.numpy as jnp
from jax import lax
from jax.experimental import pallas as pl
from jax.experimental.pallas import tpu as pltpu
from jax.experimental.pallas import tpu_sc as plsc

_NC = 1
_NS = 16
_L = 16
_NW = _NC * _NS
_B = 16384
_BPW = _B // _NW
_TABLE = 1001


def _gather_body(tab_hbm, idx_hbm, out_hbm, tab_v, idx_v, out_v, sem_t, sem_i):
    wid = lax.axis_index("s") * _NC + lax.axis_index("c")
    base = wid * _BPW
    ct = pltpu.async_copy(tab_hbm, tab_v, sem_t)
    ci = pltpu.async_copy(idx_hbm.at[pl.ds(base, _BPW)], idx_v, sem_i)
    ct.wait()
    ci.wait()
    chunk = _BPW // 4

    def step(lo):
        def body(i, carry):
            off = lo + i * _L
            idx = idx_v[pl.ds(off, _L)]
            out_v[pl.ds(off, _L)] = plsc.load_gather(tab_v, [idx])
            return carry
        return body

    copies = []
    for k in range(4):
        lax.fori_loop(0, chunk // _L, step(k * chunk), 0, unroll=4)
        copies.append(
            pltpu.async_copy(
                out_v.at[pl.ds(k * chunk, chunk)],
                out_hbm.at[pl.ds(base + k * chunk, chunk)],
                sem_i,
            )
        )
    for co in copies:
        co.wait()


def kernel(t, alpha, alpha_bar):
    del alpha
    mesh = plsc.VectorSubcoreMesh(core_axis_name="c", subcore_axis_name="s", num_cores=1)
    f = pl.kernel(
        _gather_body,
        out_type=jax.ShapeDtypeStruct((_B,), jnp.float32),
        mesh=mesh,
        scratch_types=[
            pltpu.VMEM((_TABLE,), jnp.float32),
            pltpu.VMEM((_BPW,), jnp.int32),
            pltpu.VMEM((_BPW,), jnp.float32),
            pltpu.SemaphoreType.DMA,
            pltpu.SemaphoreType.DMA,
        ],
        compiler_params=pltpu.CompilerParams(
            needs_layout_passes=False,
            disable_bounds_checks=True,
            disable_semaphore_checks=True,
            skip_device_barrier=True,
        ),
    )
    return f(alpha_bar, t)
